# Initial kernel scaffold; baseline (speedup 1.0000x reference)
import jax
import jax.numpy as jnp
from jax.experimental import pallas as pl


def kernel(feats, edge_index, edge_kernel, W_conv, conv_bias, W_lin, b_lin, ln_gamma, ln_beta):
    raise NotImplementedError("write your pallas kernel here")


# speedup vs baseline: 4.4998x; 4.4998x over previous
import functools

import jax
import jax.numpy as jnp
from jax import lax
from jax.experimental import pallas as pl
from jax.experimental.pallas import tpu as pltpu
from jax.experimental.pallas import tpu_sc as plsc

N = 10000
E = 320000
C = 128
K = 27
EPS = 1e-5

CH = 128
NCHUNK = E // CH
NWORKERS = 32
NPAD = 10240
ROWS_PER_TILE = NPAD // 16


def _mm_body(x_ref, w_ref, o_ref):
    o_ref[...] = jnp.dot(x_ref[...], w_ref[...], preferred_element_type=jnp.float32)


def _transform(feats, W2):
    BN = 400
    return pl.pallas_call(
        _mm_body,
        grid=(N // BN,),
        in_specs=[
            pl.BlockSpec((BN, C), lambda i: (i, 0)),
            pl.BlockSpec((C, K * C), lambda i: (0, 0)),
        ],
        out_specs=pl.BlockSpec((BN, K * C), lambda i: (i, 0)),
        out_shape=jax.ShapeDtypeStruct((N, K * C), jnp.float32),
    )(feats, W2)


_MESH = plsc.VectorSubcoreMesh(core_axis_name="c", subcore_axis_name="s")


@functools.partial(
    pl.kernel,
    out_type=jax.ShapeDtypeStruct((2, NPAD, C), jnp.float32),
    mesh=_MESH,
    scratch_types=[
        pltpu.VMEM((CH,), jnp.int32),
        pltpu.VMEM((CH,), jnp.int32),
        pltpu.VMEM((CH,), jnp.int32),
        pltpu.VMEM((CH,), jnp.int32),
        pltpu.VMEM((CH, C), jnp.float32),
        pltpu.VMEM_SHARED((NPAD, C), jnp.float32),
        pltpu.SemaphoreType.DMA,
    ],
)
def _sc_scatter(src_hbm, kern_hbm, dst_hbm, t_hbm, zeros_hbm, out_hbm,
                src_v, kern_v, dst_v, idx_v, rows_v, acc_sh, sem):
    cid = lax.axis_index("c")
    sid = lax.axis_index("s")
    wid = sid * 2 + cid

    pltpu.sync_copy(zeros_hbm.at[pl.ds(sid * ROWS_PER_TILE, ROWS_PER_TILE)],
                    acc_sh.at[pl.ds(sid * ROWS_PER_TILE, ROWS_PER_TILE)])
    plsc.subcore_barrier()

    n_iter = jnp.where(wid < NCHUNK - (NCHUNK // NWORKERS) * NWORKERS,
                       NCHUNK // NWORKERS + 1, NCHUNK // NWORKERS)

    def body(j, carry):
        base = (wid + j * NWORKERS) * CH
        pltpu.sync_copy(src_hbm.at[pl.ds(base, CH)], src_v)
        pltpu.sync_copy(kern_hbm.at[pl.ds(base, CH)], kern_v)
        pltpu.sync_copy(dst_hbm.at[pl.ds(base, CH)], dst_v)
        for i in range(CH // 16):
            sl = pl.ds(i * 16, 16)
            idx_v[sl] = src_v[sl] * K + kern_v[sl]
        pltpu.async_copy(t_hbm.at[idx_v], rows_v, sem).wait()
        pltpu.sync_copy(rows_v, acc_sh.at[dst_v], add=True)
        return carry

    lax.fori_loop(0, n_iter, body, 0)
    plsc.subcore_barrier()

    pltpu.sync_copy(acc_sh.at[pl.ds(sid * ROWS_PER_TILE, ROWS_PER_TILE)],
                    out_hbm.at[cid, pl.ds(sid * ROWS_PER_TILE, ROWS_PER_TILE)])


def _epi_body(p_ref, cb_ref, wl_ref, bl_ref, g_ref, b_ref, o_ref):
    conv = p_ref[0] + p_ref[1] + cb_ref[...]
    lin = lax.dot_general(conv, wl_ref[...], (((1,), (1,)), ((), ())),
                          preferred_element_type=jnp.float32) + bl_ref[...]
    mean = jnp.mean(lin, axis=1, keepdims=True)
    cent = lin - mean
    var = jnp.mean(cent * cent, axis=1, keepdims=True)
    o_ref[...] = cent * lax.rsqrt(var + EPS) * g_ref[...] + b_ref[...]


def _epilogue(partials, conv_bias, W_lin, b_lin, ln_gamma, ln_beta):
    BN = 1000
    return pl.pallas_call(
        _epi_body,
        grid=(N // BN,),
        in_specs=[
            pl.BlockSpec((2, BN, C), lambda i: (0, i, 0)),
            pl.BlockSpec((1, C), lambda i: (0, 0)),
            pl.BlockSpec((C, C), lambda i: (0, 0)),
            pl.BlockSpec((1, C), lambda i: (0, 0)),
            pl.BlockSpec((1, C), lambda i: (0, 0)),
            pl.BlockSpec((1, C), lambda i: (0, 0)),
        ],
        out_specs=pl.BlockSpec((BN, C), lambda i: (i, 0)),
        out_shape=jax.ShapeDtypeStruct((N, C), jnp.float32),
    )(partials, conv_bias.reshape(1, C), W_lin, b_lin.reshape(1, C),
      ln_gamma.reshape(1, C), ln_beta.reshape(1, C))


def kernel(feats, edge_index, edge_kernel, W_conv, conv_bias, W_lin, b_lin,
           ln_gamma, ln_beta):
    W2 = W_conv.transpose(1, 0, 2).reshape(C, K * C)
    T = _transform(feats, W2).reshape(N * K, C)
    zeros = jnp.zeros((NPAD, C), dtype=jnp.float32)
    partials = _sc_scatter(edge_index[0], edge_kernel, edge_index[1], T, zeros)
    return _epilogue(partials, conv_bias, W_lin, b_lin, ln_gamma, ln_beta)
